# baseline (speedup 1.0000x reference)
import functools

import jax
import jax.numpy as jnp
from jax import lax
from jax.experimental import pallas as pl
from jax.experimental.pallas import tpu as pltpu
from jax.experimental.pallas import tpu_sc as plsc

OUT_H = 7
OUT_W = 7
SPATIAL_SCALE = 0.125
N, C, H, W = 2, 256, 128, 128
K = 1000
KP = 1024
G = OUT_H * OUT_W
P = G * KP
NW = 32
CH = 16
LANES = 16
V = N * H * W
HALF = C // 2


def _tc_index_kernel(rt_ref, idx_ref, w_ref):
    bi = rt_ref[0:1, :].astype(jnp.int32)
    cx = rt_ref[1:2, :]
    cy = rt_ref[2:3, :]
    rw = rt_ref[3:4, :]
    rh = rt_ref[4:5, :]
    ct = rt_ref[5:6, :]
    st = rt_ref[6:7, :]
    gi = lax.broadcasted_iota(jnp.int32, (G, KP), 0)
    gxf = (gi % OUT_W).astype(jnp.float32)
    gyf = (gi // OUT_W).astype(jnp.float32)
    gx = (gxf + 0.5) / OUT_W - 0.5
    gy = (gyf + 0.5) / OUT_H - 0.5
    gxw = gx * rw
    gyh = gy * rh
    ix = gxw * ct - gyh * st + cx - 0.5
    iy = gxw * st + gyh * ct + cy - 0.5
    x0 = jnp.floor(ix)
    y0 = jnp.floor(iy)
    x1 = x0 + 1.0
    y1 = y0 + 1.0
    wx1 = ix - x0
    wx0 = 1.0 - wx1
    wy1 = iy - y0
    wy0 = 1.0 - wy1
    basex = jnp.clip(x0, 0.0, W - 2.0)
    wx0m = jnp.where((x0 >= 0.0) & (x0 <= W - 1.0), wx0, 0.0)
    wx1m = jnp.where((x1 >= 0.0) & (x1 <= W - 1.0), wx1, 0.0)
    a0 = jnp.where(x0 == basex, wx0m, 0.0) + jnp.where(x1 == basex, wx1m, 0.0)
    a1 = (jnp.where(x0 == basex + 1.0, wx0m, 0.0)
          + jnp.where(x1 == basex + 1.0, wx1m, 0.0))
    basex_i = basex.astype(jnp.int32)
    bbase = bi * (H * W)
    for r, (ycf, wyr) in enumerate(((y0, wy0), (y1, wy1))):
        validy = (ycf >= 0.0) & (ycf <= H - 1.0)
        yir = jnp.clip(ycf, 0.0, H - 1.0).astype(jnp.int32)
        s = bbase + yir * W + basex_i
        idx_ref[r] = (s >> 1) + (s & 1) * (V // 2)
        br = jnp.where(validy, wyr, 0.0)
        w_ref[2 * r] = br * a0
        w_ref[2 * r + 1] = br * a1


_tc_index = pl.pallas_call(
    _tc_index_kernel,
    out_shape=(
        jax.ShapeDtypeStruct((2, G, KP), jnp.int32),
        jax.ShapeDtypeStruct((4, G, KP), jnp.float32),
    ),
)


def _make_sc_gather(nw):
    per_w = P // nw
    n_chunks = per_w // CH
    assert n_chunks % 2 == 0
    mesh = plsc.VectorSubcoreMesh(core_axis_name="c", subcore_axis_name="s")

    @functools.partial(
        pl.kernel,
        mesh=mesh,
        out_type=jax.ShapeDtypeStruct((P, C), jnp.float32),
        scratch_types=[
            pltpu.VMEM((n_chunks, 1, 2 * CH), jnp.int32),
            pltpu.VMEM((2, 4 * CH, LANES), jnp.float32),
            pltpu.VMEM((2, 2 * CH, C), jnp.uint32),
            pltpu.VMEM((2, CH, C), jnp.float32),
            pltpu.SemaphoreType.DMA,
            pltpu.SemaphoreType.DMA,
            pltpu.SemaphoreType.DMA,
            pltpu.SemaphoreType.DMA,
            pltpu.SemaphoreType.DMA,
            pltpu.SemaphoreType.DMA,
        ],
    )
    def sc_fn(feats_hbm, idx_hbm, w_hbm, out_hbm, idx_all, w_v, rows_v, out_v,
              gs0, gs1, ws0, ws1, os0, os1):
        gsems = (gs0, gs1)
        wsems = (ws0, ws1)
        osems = (os0, os1)
        wid = lax.axis_index("s") * 2 + lax.axis_index("c")
        cbase = wid * n_chunks
        pbase = wid * per_w
        pltpu.sync_copy(idx_hbm.at[wid], idx_all)

        def start(ck, b):
            pltpu.async_copy(w_hbm.at[cbase + ck], w_v.at[b], wsems[b])
            pltpu.async_copy(feats_hbm.at[idx_all.at[ck, 0]], rows_v.at[b],
                             gsems[b])

        def compute(ck, b):
            pltpu.make_async_copy(w_hbm.at[cbase + ck], w_v.at[b],
                                  wsems[b]).wait()
            pltpu.make_async_copy(
                feats_hbm.at[idx_all.at[ck, 0]], rows_v.at[b], gsems[b]).wait()

            @pl.when(ck >= 2)
            def _():
                pltpu.make_async_copy(
                    out_v.at[b], out_hbm.at[pl.ds(0, CH)], osems[b]).wait()

            def pbody(p, c2):
                wb = (w_v[b, p, :], w_v[b, CH + p, :],
                      w_v[b, 2 * CH + p, :], w_v[b, 3 * CH + p, :])
                r1 = CH + p
                hm = jnp.uint32(0xFFFF0000)
                for g in range(C // (2 * LANES)):
                    sa = pl.ds(g * LANES, LANES)
                    sb = pl.ds(HALF + g * LANES, LANES)
                    u00 = rows_v[b, p, sa]
                    u01 = rows_v[b, p, sb]
                    u10 = rows_v[b, r1, sa]
                    u11 = rows_v[b, r1, sb]
                    acc_a = lax.bitcast_convert_type(u00 << 16, jnp.float32) * wb[0]
                    acc_b = lax.bitcast_convert_type(u00 & hm, jnp.float32) * wb[0]
                    acc_a += lax.bitcast_convert_type(u01 << 16, jnp.float32) * wb[1]
                    acc_b += lax.bitcast_convert_type(u01 & hm, jnp.float32) * wb[1]
                    acc_a += lax.bitcast_convert_type(u10 << 16, jnp.float32) * wb[2]
                    acc_b += lax.bitcast_convert_type(u10 & hm, jnp.float32) * wb[2]
                    acc_a += lax.bitcast_convert_type(u11 << 16, jnp.float32) * wb[3]
                    acc_b += lax.bitcast_convert_type(u11 & hm, jnp.float32) * wb[3]
                    out_v[b, p, sa] = acc_a
                    out_v[b, p, sb] = acc_b
                return c2

            lax.fori_loop(0, CH, pbody, 0)
            pltpu.async_copy(
                out_v.at[b], out_hbm.at[pl.ds(pbase + ck * CH, CH)], osems[b])

        start(0, 0)

        def pair(g, carry):
            ck = 2 * g
            start(ck + 1, 1)
            compute(ck, 0)
            start(ck + 2, 0)
            compute(ck + 1, 1)
            return carry

        lax.fori_loop(0, (n_chunks - 2) // 2, pair, 0)
        start(n_chunks - 1, 1)
        compute(n_chunks - 2, 0)
        compute(n_chunks - 1, 1)
        pltpu.make_async_copy(out_v.at[0], out_hbm.at[pl.ds(0, CH)], osems[0]).wait()
        pltpu.make_async_copy(out_v.at[1], out_hbm.at[pl.ds(0, CH)], osems[1]).wait()

    return sc_fn


@functools.cache
def _sc_gather_cached():
    return _make_sc_gather(NW)


def _pack_pairs(u):
    lo = u[..., 0, :]
    hi = u[..., 1, :]
    one = jnp.uint32(1)
    r = jnp.uint32(0x7FFF)
    lo16 = (lo + r + ((lo >> 16) & one)) >> 16
    hi16 = (((hi + r + ((hi >> 16) & one)) >> 16) << 16)
    return lo16 | hi16


def kernel(features, rois):
    t = jnp.transpose(features, (0, 2, 3, 1)).reshape(V, C)
    u = lax.bitcast_convert_type(t, jnp.uint32).reshape(V, 2, HALF)
    packed = _pack_pairs(u)
    tp_even = packed.reshape(V // 2, C)
    tp_odd = packed[1:V - 1].reshape(V // 2 - 1, C)
    tp = jnp.concatenate([tp_even, tp_odd], axis=0)

    th = rois[:, 5] * SPATIAL_SCALE
    rt = jnp.stack(
        [
            rois[:, 0],
            rois[:, 1] * SPATIAL_SCALE,
            rois[:, 2] * SPATIAL_SCALE,
            rois[:, 3] * SPATIAL_SCALE,
            rois[:, 4] * SPATIAL_SCALE,
            jnp.cos(th),
            jnp.sin(th),
            jnp.zeros_like(th),
        ],
        axis=0,
    )
    rt = jnp.pad(rt, ((0, 0), (0, KP - K)))
    idx2, w4 = _tc_index(rt)
    nck = P // CH
    idx_sc = (
        idx2.reshape(2, nck, CH).transpose(1, 0, 2)
        .reshape(NW, nck // NW, 1, 2 * CH)
    )
    w_sc = w4.reshape(4, nck, CH).transpose(1, 0, 2).reshape(nck, 4 * CH)
    w_sc = jnp.broadcast_to(w_sc[:, :, None], (nck, 4 * CH, LANES))
    out2 = _sc_gather_cached()(tp, idx_sc, w_sc)
    out = out2.reshape(G, KP, C)[:, :K]
    return out.transpose(1, 2, 0).reshape(K, C, OUT_H, OUT_W)


# speedup vs baseline: 1.0387x; 1.0387x over previous
import functools

import jax
import jax.numpy as jnp
from jax import lax
from jax.experimental import pallas as pl
from jax.experimental.pallas import tpu as pltpu
from jax.experimental.pallas import tpu_sc as plsc

OUT_H = 7
OUT_W = 7
SPATIAL_SCALE = 0.125
N, C, H, W = 2, 256, 128, 128
K = 1000
KP = 1024
G = OUT_H * OUT_W
P = G * KP
NW = 32
CH = 16
LANES = 16
V = N * H * W
HALF = C // 2


def _tc_index_kernel(rt_ref, idx_ref, w_ref):
    bi = rt_ref[0:1, :].astype(jnp.int32)
    cx = rt_ref[1:2, :]
    cy = rt_ref[2:3, :]
    rw = rt_ref[3:4, :]
    rh = rt_ref[4:5, :]
    ct = rt_ref[5:6, :]
    st = rt_ref[6:7, :]
    gi = lax.broadcasted_iota(jnp.int32, (G, KP), 0)
    gxf = (gi % OUT_W).astype(jnp.float32)
    gyf = (gi // OUT_W).astype(jnp.float32)
    gx = (gxf + 0.5) / OUT_W - 0.5
    gy = (gyf + 0.5) / OUT_H - 0.5
    gxw = gx * rw
    gyh = gy * rh
    ix = gxw * ct - gyh * st + cx - 0.5
    iy = gxw * st + gyh * ct + cy - 0.5
    x0 = jnp.floor(ix)
    y0 = jnp.floor(iy)
    x1 = x0 + 1.0
    y1 = y0 + 1.0
    wx1 = ix - x0
    wx0 = 1.0 - wx1
    wy1 = iy - y0
    wy0 = 1.0 - wy1
    basex = jnp.clip(x0, 0.0, W - 2.0)
    wx0m = jnp.where((x0 >= 0.0) & (x0 <= W - 1.0), wx0, 0.0)
    wx1m = jnp.where((x1 >= 0.0) & (x1 <= W - 1.0), wx1, 0.0)
    a0 = jnp.where(x0 == basex, wx0m, 0.0) + jnp.where(x1 == basex, wx1m, 0.0)
    a1 = (jnp.where(x0 == basex + 1.0, wx0m, 0.0)
          + jnp.where(x1 == basex + 1.0, wx1m, 0.0))
    basex_i = basex.astype(jnp.int32)
    bbase = bi * (H * W)
    for r, (ycf, wyr) in enumerate(((y0, wy0), (y1, wy1))):
        validy = (ycf >= 0.0) & (ycf <= H - 1.0)
        yir = jnp.clip(ycf, 0.0, H - 1.0).astype(jnp.int32)
        s = bbase + yir * W + basex_i
        idx_ref[r] = (s >> 1) + (s & 1) * (V // 2)
        br = jnp.where(validy, wyr, 0.0)
        w_ref[2 * r] = br * a0
        w_ref[2 * r + 1] = br * a1


_tc_index = pl.pallas_call(
    _tc_index_kernel,
    out_shape=(
        jax.ShapeDtypeStruct((2, G, KP), jnp.int32),
        jax.ShapeDtypeStruct((4, G, KP), jnp.float32),
    ),
)


def _make_sc_gather(nw):
    per_w = P // nw
    n_chunks = per_w // CH
    assert n_chunks % 2 == 0
    mesh = plsc.VectorSubcoreMesh(core_axis_name="c", subcore_axis_name="s")

    @functools.partial(
        pl.kernel,
        mesh=mesh,
        out_type=jax.ShapeDtypeStruct((P, HALF), jnp.uint32),
        scratch_types=[
            pltpu.VMEM((n_chunks, 1, 2 * CH), jnp.int32),
            pltpu.VMEM((n_chunks, 1, 4 * CH), jnp.float32),
            pltpu.VMEM((2, 2 * CH, C), jnp.uint32),
            pltpu.VMEM((2, CH, HALF), jnp.uint32),
            pltpu.SemaphoreType.DMA,
            pltpu.SemaphoreType.DMA,
            pltpu.SemaphoreType.DMA,
            pltpu.SemaphoreType.DMA,
        ],
    )
    def sc_fn(feats_hbm, idx_hbm, w_hbm, out_hbm, idx_all, w_all, rows_v, out_v,
              gs0, gs1, os0, os1):
        gsems = (gs0, gs1)
        osems = (os0, os1)
        wid = lax.axis_index("s") * 2 + lax.axis_index("c")
        pbase = wid * per_w
        pltpu.sync_copy(idx_hbm.at[wid], idx_all)
        pltpu.sync_copy(w_hbm.at[wid], w_all)

        def start(ck, b):
            pltpu.async_copy(feats_hbm.at[idx_all.at[ck, 0]], rows_v.at[b],
                             gsems[b])

        def compute(ck, b):
            pltpu.make_async_copy(
                feats_hbm.at[idx_all.at[ck, 0]], rows_v.at[b], gsems[b]).wait()

            @pl.when(ck >= 2)
            def _():
                pltpu.make_async_copy(
                    out_v.at[b], out_hbm.at[pl.ds(0, CH)], osems[b]).wait()

            w0 = w_all[ck, 0, pl.ds(0, LANES)]
            w1 = w_all[ck, 0, pl.ds(CH, LANES)]
            w2 = w_all[ck, 0, pl.ds(2 * CH, LANES)]
            w3 = w_all[ck, 0, pl.ds(3 * CH, LANES)]

            def pbody(p, c2):
                lane = jnp.zeros((LANES,), jnp.int32) + p
                wb = tuple(
                    wv.at[lane].get(mode="promise_in_bounds")
                    for wv in (w0, w1, w2, w3))
                r1 = CH + p
                hm = jnp.uint32(0xFFFF0000)
                for g in range(C // (2 * LANES)):
                    sa = pl.ds(g * LANES, LANES)
                    sb = pl.ds(HALF + g * LANES, LANES)
                    u00 = rows_v[b, p, sa]
                    u01 = rows_v[b, p, sb]
                    u10 = rows_v[b, r1, sa]
                    u11 = rows_v[b, r1, sb]
                    acc_a = lax.bitcast_convert_type(u00 << 16, jnp.float32) * wb[0]
                    acc_b = lax.bitcast_convert_type(u00 & hm, jnp.float32) * wb[0]
                    acc_a += lax.bitcast_convert_type(u01 << 16, jnp.float32) * wb[1]
                    acc_b += lax.bitcast_convert_type(u01 & hm, jnp.float32) * wb[1]
                    acc_a += lax.bitcast_convert_type(u10 << 16, jnp.float32) * wb[2]
                    acc_b += lax.bitcast_convert_type(u10 & hm, jnp.float32) * wb[2]
                    acc_a += lax.bitcast_convert_type(u11 << 16, jnp.float32) * wb[3]
                    acc_b += lax.bitcast_convert_type(u11 & hm, jnp.float32) * wb[3]
                    ua = lax.bitcast_convert_type(acc_a, jnp.uint32) >> 16
                    ub = lax.bitcast_convert_type(acc_b, jnp.uint32) & hm
                    out_v[b, p, sa] = ua | ub
                return c2

            lax.fori_loop(0, CH, pbody, 0)
            pltpu.async_copy(
                out_v.at[b], out_hbm.at[pl.ds(pbase + ck * CH, CH)], osems[b])

        start(0, 0)

        def pair(g, carry):
            ck = 2 * g
            start(ck + 1, 1)
            compute(ck, 0)
            start(ck + 2, 0)
            compute(ck + 1, 1)
            return carry

        lax.fori_loop(0, (n_chunks - 2) // 2, pair, 0)
        start(n_chunks - 1, 1)
        compute(n_chunks - 2, 0)
        compute(n_chunks - 1, 1)
        pltpu.make_async_copy(out_v.at[0], out_hbm.at[pl.ds(0, CH)], osems[0]).wait()
        pltpu.make_async_copy(out_v.at[1], out_hbm.at[pl.ds(0, CH)], osems[1]).wait()

    return sc_fn


@functools.cache
def _sc_gather_cached():
    return _make_sc_gather(NW)


def _pack_pairs(u):
    lo = u[..., 0, :]
    hi = u[..., 1, :]
    one = jnp.uint32(1)
    r = jnp.uint32(0x7FFF)
    lo16 = (lo + r + ((lo >> 16) & one)) >> 16
    hi16 = (((hi + r + ((hi >> 16) & one)) >> 16) << 16)
    return lo16 | hi16


def kernel(features, rois):
    t = jnp.transpose(features, (0, 2, 3, 1)).reshape(V, C)
    u = lax.bitcast_convert_type(t, jnp.uint32).reshape(V, 2, HALF)
    packed = _pack_pairs(u)
    tp_even = packed.reshape(V // 2, C)
    tp_odd = packed[1:V - 1].reshape(V // 2 - 1, C)
    tp = jnp.concatenate([tp_even, tp_odd], axis=0)

    th = rois[:, 5] * SPATIAL_SCALE
    rt = jnp.stack(
        [
            rois[:, 0],
            rois[:, 1] * SPATIAL_SCALE,
            rois[:, 2] * SPATIAL_SCALE,
            rois[:, 3] * SPATIAL_SCALE,
            rois[:, 4] * SPATIAL_SCALE,
            jnp.cos(th),
            jnp.sin(th),
            jnp.zeros_like(th),
        ],
        axis=0,
    )
    rt = jnp.pad(rt, ((0, 0), (0, KP - K)))
    idx2, w4 = _tc_index(rt)
    nck = P // CH
    idx_sc = (
        idx2.reshape(2, nck, CH).transpose(1, 0, 2)
        .reshape(NW, nck // NW, 1, 2 * CH)
    )
    w_sc = (
        w4.reshape(4, nck, CH).transpose(1, 0, 2)
        .reshape(NW, nck // NW, 1, 4 * CH)
    )
    outp = _sc_gather_cached()(tp, idx_sc, w_sc)
    lo = lax.bitcast_convert_type(outp << 16, jnp.float32)
    hi = lax.bitcast_convert_type(outp & jnp.uint32(0xFFFF0000), jnp.float32)
    out = jnp.concatenate([lo, hi], axis=-1)
    out = out.reshape(G, KP, C)[:, :K]
    return out.transpose(1, 2, 0).reshape(K, C, OUT_H, OUT_W)


# speedup vs baseline: 1.1122x; 1.0708x over previous
import functools

import jax
import jax.numpy as jnp
from jax import lax
from jax.experimental import pallas as pl
from jax.experimental.pallas import tpu as pltpu
from jax.experimental.pallas import tpu_sc as plsc

OUT_H = 7
OUT_W = 7
SPATIAL_SCALE = 0.125
N, C, H, W = 2, 256, 128, 128
K = 1000
KP = 1024
G = OUT_H * OUT_W
P = G * KP
NW = 32
CH = 16
LANES = 16
V = N * H * W
HALF = C // 2


def _tc_index_kernel(rt_ref, idx_ref, w_ref):
    bi = rt_ref[0:1, :].astype(jnp.int32)
    cx = rt_ref[1:2, :]
    cy = rt_ref[2:3, :]
    rw = rt_ref[3:4, :]
    rh = rt_ref[4:5, :]
    ct = rt_ref[5:6, :]
    st = rt_ref[6:7, :]
    gi = lax.broadcasted_iota(jnp.int32, (G, KP), 0)
    gxf = (gi % OUT_W).astype(jnp.float32)
    gyf = (gi // OUT_W).astype(jnp.float32)
    gx = (gxf + 0.5) / OUT_W - 0.5
    gy = (gyf + 0.5) / OUT_H - 0.5
    gxw = gx * rw
    gyh = gy * rh
    ix = gxw * ct - gyh * st + cx - 0.5
    iy = gxw * st + gyh * ct + cy - 0.5
    x0 = jnp.floor(ix)
    y0 = jnp.floor(iy)
    x1 = x0 + 1.0
    y1 = y0 + 1.0
    wx1 = ix - x0
    wx0 = 1.0 - wx1
    wy1 = iy - y0
    wy0 = 1.0 - wy1
    basex = jnp.clip(x0, 0.0, W - 2.0)
    wx0m = jnp.where((x0 >= 0.0) & (x0 <= W - 1.0), wx0, 0.0)
    wx1m = jnp.where((x1 >= 0.0) & (x1 <= W - 1.0), wx1, 0.0)
    a0 = jnp.where(x0 == basex, wx0m, 0.0) + jnp.where(x1 == basex, wx1m, 0.0)
    a1 = (jnp.where(x0 == basex + 1.0, wx0m, 0.0)
          + jnp.where(x1 == basex + 1.0, wx1m, 0.0))
    basex_i = basex.astype(jnp.int32)
    bbase = bi * (H * W)
    for r, (ycf, wyr) in enumerate(((y0, wy0), (y1, wy1))):
        validy = (ycf >= 0.0) & (ycf <= H - 1.0)
        yir = jnp.clip(ycf, 0.0, H - 1.0).astype(jnp.int32)
        s = bbase + yir * W + basex_i
        idx_ref[r] = (s >> 1) + (s & 1) * (V // 2)
        br = jnp.where(validy, wyr, 0.0)
        w_ref[2 * r] = br * a0
        w_ref[2 * r + 1] = br * a1


_tc_index = pl.pallas_call(
    _tc_index_kernel,
    out_shape=(
        jax.ShapeDtypeStruct((2, G, KP), jnp.int32),
        jax.ShapeDtypeStruct((4, G, KP), jnp.float32),
    ),
)


def _make_sc_gather(nw):
    per_w = P // nw
    n_chunks = per_w // CH
    assert n_chunks % 2 == 0
    mesh = plsc.VectorSubcoreMesh(core_axis_name="c", subcore_axis_name="s")

    @functools.partial(
        pl.kernel,
        mesh=mesh,
        out_type=jax.ShapeDtypeStruct((P, HALF), jnp.uint32),
        scratch_types=[
            pltpu.VMEM((n_chunks, 1, 2 * CH), jnp.int32),
            pltpu.VMEM((n_chunks, 1, 4 * CH), jnp.float32),
            pltpu.VMEM((2, 2 * CH, C), jnp.uint32),
            pltpu.VMEM((2, CH, HALF), jnp.uint32),
            pltpu.SemaphoreType.DMA,
            pltpu.SemaphoreType.DMA,
            pltpu.SemaphoreType.DMA,
            pltpu.SemaphoreType.DMA,
        ],
    )
    def sc_fn(feats_hbm, idx_hbm, w_hbm, out_hbm, idx_all, w_all, rows_v, out_v,
              gs0, gs1, os0, os1):
        gsems = (gs0, gs1)
        osems = (os0, os1)
        wid = lax.axis_index("s") * 2 + lax.axis_index("c")
        pbase = wid * per_w
        pltpu.sync_copy(idx_hbm.at[wid], idx_all)
        pltpu.sync_copy(w_hbm.at[wid], w_all)

        def start(ck, b):
            pltpu.async_copy(feats_hbm.at[idx_all.at[ck, 0]], rows_v.at[b],
                             gsems[b])

        def compute(ck, b):
            pltpu.make_async_copy(
                feats_hbm.at[idx_all.at[ck, 0]], rows_v.at[b], gsems[b]).wait()

            @pl.when(ck >= 2)
            def _():
                pltpu.make_async_copy(
                    out_v.at[b], out_hbm.at[pl.ds(0, CH)], osems[b]).wait()

            w0 = w_all[ck, 0, pl.ds(0, LANES)]
            w1 = w_all[ck, 0, pl.ds(CH, LANES)]
            w2 = w_all[ck, 0, pl.ds(2 * CH, LANES)]
            w3 = w_all[ck, 0, pl.ds(3 * CH, LANES)]

            def pbody(p, c2):
                lane = jnp.zeros((LANES,), jnp.int32) + p
                wb = tuple(
                    wv.at[lane].get(mode="promise_in_bounds")
                    for wv in (w0, w1, w2, w3))
                r1 = CH + p
                hm = jnp.uint32(0xFFFF0000)
                for g in range(C // (2 * LANES)):
                    sa = pl.ds(g * LANES, LANES)
                    sb = pl.ds(HALF + g * LANES, LANES)
                    u00 = rows_v[b, p, sa]
                    u01 = rows_v[b, p, sb]
                    u10 = rows_v[b, r1, sa]
                    u11 = rows_v[b, r1, sb]
                    acc_a = lax.bitcast_convert_type(u00 << 16, jnp.float32) * wb[0]
                    acc_b = lax.bitcast_convert_type(u00 & hm, jnp.float32) * wb[0]
                    acc_a += lax.bitcast_convert_type(u01 << 16, jnp.float32) * wb[1]
                    acc_b += lax.bitcast_convert_type(u01 & hm, jnp.float32) * wb[1]
                    acc_a += lax.bitcast_convert_type(u10 << 16, jnp.float32) * wb[2]
                    acc_b += lax.bitcast_convert_type(u10 & hm, jnp.float32) * wb[2]
                    acc_a += lax.bitcast_convert_type(u11 << 16, jnp.float32) * wb[3]
                    acc_b += lax.bitcast_convert_type(u11 & hm, jnp.float32) * wb[3]
                    ua = lax.bitcast_convert_type(acc_a, jnp.uint32) >> 16
                    ub = lax.bitcast_convert_type(acc_b, jnp.uint32) & hm
                    out_v[b, p, sa] = ua | ub
                return c2

            lax.fori_loop(0, CH, pbody, 0)
            pltpu.async_copy(
                out_v.at[b], out_hbm.at[pl.ds(pbase + ck * CH, CH)], osems[b])

        start(0, 0)

        def pair(g, carry):
            ck = 2 * g
            start(ck + 1, 1)
            compute(ck, 0)
            start(ck + 2, 0)
            compute(ck + 1, 1)
            return carry

        lax.fori_loop(0, (n_chunks - 2) // 2, pair, 0)
        start(n_chunks - 1, 1)
        compute(n_chunks - 2, 0)
        compute(n_chunks - 1, 1)
        pltpu.make_async_copy(out_v.at[0], out_hbm.at[pl.ds(0, CH)], osems[0]).wait()
        pltpu.make_async_copy(out_v.at[1], out_hbm.at[pl.ds(0, CH)], osems[1]).wait()

    return sc_fn


@functools.cache
def _sc_gather_cached():
    return _make_sc_gather(NW)


def _pack_pairs(u):
    lo = u[..., 0, :]
    hi = u[..., 1, :]
    one = jnp.uint32(1)
    r = jnp.uint32(0x7FFF)
    lo16 = (lo + r + ((lo >> 16) & one)) >> 16
    hi16 = (((hi + r + ((hi >> 16) & one)) >> 16) << 16)
    return lo16 | hi16


def kernel(features, rois):
    t = jnp.transpose(features, (0, 2, 3, 1)).reshape(V, C)
    u = lax.bitcast_convert_type(t, jnp.uint32).reshape(V, 2, HALF)
    packed = _pack_pairs(u)
    tp_even = packed.reshape(V // 2, C)
    tp_odd = packed[1:V - 1].reshape(V // 2 - 1, C)
    tp = jnp.concatenate([tp_even, tp_odd], axis=0)

    th = rois[:, 5] * SPATIAL_SCALE
    rt = jnp.stack(
        [
            rois[:, 0],
            rois[:, 1] * SPATIAL_SCALE,
            rois[:, 2] * SPATIAL_SCALE,
            rois[:, 3] * SPATIAL_SCALE,
            rois[:, 4] * SPATIAL_SCALE,
            jnp.cos(th),
            jnp.sin(th),
            jnp.zeros_like(th),
        ],
        axis=0,
    )
    rt = jnp.pad(rt, ((0, 0), (0, KP - K)))
    idx2, w4 = _tc_index(rt)
    nck = P // CH
    idx_sc = (
        idx2.reshape(2, nck, CH).transpose(1, 0, 2)
        .reshape(NW, nck // NW, 1, 2 * CH)
    )
    w_sc = (
        w4.reshape(4, nck, CH).transpose(1, 0, 2)
        .reshape(NW, nck // NW, 1, 4 * CH)
    )
    outp = _sc_gather_cached()(tp, idx_sc, w_sc)
    ot = outp.reshape(G, KP, HALF)[:, :K].transpose(1, 2, 0)
    lo = lax.bitcast_convert_type(ot << 16, jnp.float32)
    hi = lax.bitcast_convert_type(ot & jnp.uint32(0xFFFF0000), jnp.float32)
    out = jnp.concatenate([lo, hi], axis=1)
    return out.reshape(K, C, OUT_H, OUT_W)
